# baseline (speedup 1.0000x reference)
import jax
import jax.numpy as jnp
from jax import lax
from jax.experimental import pallas as pl

N = 10000
E = 320000
LAT = 16
G = 8
F = 3 * LAT + G
DT = 0.01
STEPS = 8

EB = 8000


def _edge_enc_body(e_ref, w1_ref, b1_ref, w2_ref, b2_ref, out_ref):
    x = e_ref[...]
    h = jnp.dot(x, w1_ref[...], preferred_element_type=jnp.float32) + b1_ref[...]
    h = h * jax.nn.sigmoid(h)
    out_ref[...] = (
        jnp.dot(h, w2_ref[...], preferred_element_type=jnp.float32) + b2_ref[...]
    )


def _edge_encoder(edges, w1, b1, w2, b2):
    return pl.pallas_call(
        _edge_enc_body,
        grid=(E // EB,),
        in_specs=[
            pl.BlockSpec((EB, 4), lambda i: (i, 0)),
            pl.BlockSpec((4, LAT), lambda i: (0, 0)),
            pl.BlockSpec((1, LAT), lambda i: (0, 0)),
            pl.BlockSpec((LAT, LAT), lambda i: (0, 0)),
            pl.BlockSpec((1, LAT), lambda i: (0, 0)),
        ],
        out_specs=pl.BlockSpec((EB, LAT), lambda i: (i, 0)),
        out_shape=jax.ShapeDtypeStruct((E, LAT), jnp.float32),
    )(edges, w1, b1.reshape(1, LAT), w2, b2.reshape(1, LAT))


def _main_body(nodes_ref, sent_ref, recv_ref, gl_ref,
               wn1_ref, bn1_ref, wn2_ref, bn2_ref,
               w1k_ref, b1eff_ref, w2_ref, b2_ref,
               procw_ref, procb_ref,
               d1_ref, db1_ref, d2_ref, db2_ref, d3_ref, db3_ref,
               nn_ref, ne_ref, gn_ref):
    nodes = nodes_ref[...]
    h = jnp.dot(nodes, wn1_ref[...], preferred_element_type=jnp.float32) + bn1_ref[...]
    h = h * jax.nn.sigmoid(h)
    hn = jnp.dot(h, wn2_ref[...], preferred_element_type=jnp.float32) + bn2_ref[...]
    sent = sent_ref[...]
    recv = recv_ref[...]
    gl = jnp.broadcast_to(gl_ref[...], (N, G))
    y = jnp.concatenate([hn, sent, recv, gl], axis=1)

    W1 = w1k_ref[...]
    W2 = w2_ref[...]
    b2 = b2_ref[...]
    hstep = 1.0 / STEPS

    def f(yy, bvec):
        t = jnp.dot(yy, W1, preferred_element_type=jnp.float32) + bvec
        t = jnp.maximum(t, 0.0)
        return jnp.dot(t, W2, preferred_element_type=jnp.float32) + b2

    for i in range(STEPS):
        ba = b1eff_ref[3 * i:3 * i + 1, :]
        bm = b1eff_ref[3 * i + 1:3 * i + 2, :]
        bc = b1eff_ref[3 * i + 2:3 * i + 3, :]
        k1 = f(y, ba)
        k2 = f(y + (0.5 * hstep) * k1, bm)
        k3 = f(y + (0.5 * hstep) * k2, bm)
        k4 = f(y + hstep * k3, bc)
        y = y + (hstep / 6.0) * (k1 + 2.0 * k2 + 2.0 * k3 + k4)

    hn2 = jnp.dot(y, procw_ref[...], preferred_element_type=jnp.float32) + procb_ref[...]
    d = jnp.dot(hn2, d1_ref[...], preferred_element_type=jnp.float32) + db1_ref[...]
    d = d * jax.nn.sigmoid(d)
    d = jnp.dot(d, d2_ref[...], preferred_element_type=jnp.float32) + db2_ref[...]
    d = d * jax.nn.sigmoid(d)
    acc = jnp.dot(d, d3_ref[...], preferred_element_type=jnp.float32) + db3_ref[...]

    nv = nodes[:, 127:128] + acc * DT
    npos = nodes[:, 0:1] + nv * DT
    nn_ref[...] = jnp.concatenate([npos, nodes[:, 2:], nv, acc], axis=1)
    ne_ref[...] = npos[1:, :] - npos[:-1, :]
    g = gl_ref[...]
    gn_ref[...] = jnp.concatenate([g[:, :1] + 1.0, g[:, 1:]], axis=1)


def _main(nodes, sent, recv, gl2, p, b1eff, w1k):
    return pl.pallas_call(
        _main_body,
        out_shape=[
            jax.ShapeDtypeStruct((N, 129), jnp.float32),
            jax.ShapeDtypeStruct((N - 1, 1), jnp.float32),
            jax.ShapeDtypeStruct((1, G), jnp.float32),
        ],
    )(nodes, sent, recv, gl2,
      p['enc_n_W1'], p['enc_n_b1'].reshape(1, LAT),
      p['enc_n_W2'], p['enc_n_b2'].reshape(1, LAT),
      w1k, b1eff, p['ode_W2'], p['ode_b2'].reshape(1, F),
      p['proc_W'], p['proc_b'].reshape(1, LAT),
      p['dec_n_W1'], p['dec_n_b1'].reshape(1, LAT),
      p['dec_n_W2'], p['dec_n_b2'].reshape(1, LAT),
      p['dec_n_W3'], p['dec_n_b3'].reshape(1, 1))


def kernel(nodes, edges, globals_, params, senders, receivers):
    p = params
    he = _edge_encoder(edges, p['enc_e_W1'], p['enc_e_b1'],
                       p['enc_e_W2'], p['enc_e_b2'])
    sent = jax.ops.segment_sum(he, senders, num_segments=N)
    recv = jax.ops.segment_sum(he, receivers, num_segments=N)

    w1_full = p['ode_W1']
    w1k = w1_full[:F]
    wt = w1_full[F:F + 1]
    hstep = 1.0 / STEPS
    ts = []
    for i in range(STEPS):
        t = i * hstep
        ts.extend([t, t + 0.5 * hstep, t + hstep])
    tvec = jnp.asarray(ts, jnp.float32).reshape(-1, 1)
    b1eff = p['ode_b1'].reshape(1, F) + tvec * wt

    gl2 = globals_.reshape(1, G)
    nn, ne, gn = _main(nodes, sent, recv, gl2, p, b1eff, w1k)
    return nn, ne, gn.reshape(G)
